# baseline (speedup 1.0000x reference)
import functools

import jax
import jax.numpy as jnp
from jax.experimental import pallas as pl
from jax.experimental.pallas import tpu as pltpu

_S = 3.0
_M = 0.2
_NSPLIT = 4
_RSUB = 8


def _panel_loss(x, t, R, C):
    m = jnp.max(x, axis=1, keepdims=True)
    s = jnp.sum(jnp.exp(_S * x - _S * m), axis=1, keepdims=True)
    cols = jax.lax.broadcasted_iota(jnp.int32, (R, C), 1)
    tv = jnp.sum(jnp.where(cols == t, x, 0.0), axis=1, keepdims=True)
    out_t = _S * tv - _S * _M
    s_c = s - jnp.exp(_S * tv - _S * m) + jnp.exp(out_t - _S * m)
    return jnp.sum(_S * m + jnp.log(s_c) - out_t)


def _loss_kernel(x0, x1, x2, x3, t_ref, out_ref, *, C, B):
    i = pl.program_id(0)

    @pl.when(i == 0)
    def _init():
        out_ref[...] = jnp.zeros((1, 1), jnp.float32)

    t = t_ref[...]
    acc = 0.0
    for q, x in enumerate((x0, x1, x2, x3)):
        acc += _panel_loss(
            x[...], t[q * _RSUB : (q + 1) * _RSUB, :], _RSUB, C
        )
    out_ref[...] += (acc / B).reshape(1, 1)


def kernel(cos_theta, cos_theta_aux, target):
    B, C = cos_theta.shape
    R = _NSPLIT * _RSUB

    def make_spec(q):
        return pl.BlockSpec((_RSUB, C), lambda i, q=q: (_NSPLIT * i + q, 0))

    out = pl.pallas_call(
        functools.partial(_loss_kernel, C=C, B=B),
        grid=(B // R,),
        in_specs=[make_spec(q) for q in range(_NSPLIT)]
        + [pl.BlockSpec((R, 1), lambda i: (i, 0))],
        out_specs=pl.BlockSpec((1, 1), lambda i: (0, 0)),
        out_shape=jax.ShapeDtypeStruct((1, 1), jnp.float32),
    )(*([cos_theta] * _NSPLIT), target.reshape(B, 1).astype(jnp.int32))
    return out[0, 0]
